# Initial kernel scaffold; baseline (speedup 1.0000x reference)
import jax
import jax.numpy as jnp
from jax.experimental import pallas as pl


def kernel(x, edge_index, batch, W1, b1, W2, b2, W3, b3, fc1_W, fc1_b, fc2_W, fc2_b):
    raise NotImplementedError("write your pallas kernel here")


# speedup vs baseline: 7.6170x; 7.6170x over previous
import functools

import jax
import jax.numpy as jnp
from jax import lax
from jax.experimental import pallas as pl
from jax.experimental.pallas import tpu as pltpu
from jax.experimental.pallas import tpu_sc as plsc

N_NODES = 100000
N_EDGES = 1600000
NUM_GRAPHS = 128

W8 = 8
NB = 1024
N_PAD = 100352
STRIPE = N_PAD // 16
ZCHUNK = STRIPE // 8

NTILES = 32
EROW = 128
TILE_ROWS = 392
EROWS = NTILES * TILE_ROWS
BB = 8
NBLK = TILE_ROWS // BB
DUMP_ROW = N_PAD - 1


def _sc_agg(n_pass):
    mesh = plsc.VectorSubcoreMesh(core_axis_name="c", subcore_axis_name="s")
    scratch = [
        pltpu.VMEM((BB, EROW), jnp.int32),
        pltpu.VMEM((BB, EROW), jnp.int32),
        pltpu.VMEM((BB, EROW), jnp.int32),
        pltpu.VMEM((BB * EROW, W8), jnp.float32),
        pltpu.VMEM((ZCHUNK, W8), jnp.float32),
        pltpu.VMEM_SHARED((N_PAD, W8), jnp.float32),
        pltpu.SemaphoreType.DMA,
    ]

    @functools.partial(
        pl.kernel,
        mesh=mesh,
        out_type=jax.ShapeDtypeStruct((2, n_pass, N_PAD, W8), jnp.float32),
        scratch_types=scratch,
        compiler_params=pltpu.CompilerParams(use_tc_tiling_on_sc=False),
    )
    def k(table_hbm, src_hbm, dst_hbm, out_hbm,
          src_v, dst_v, idx_v, rows_v, zero_v, acc_sh, sem):
        cid = lax.axis_index("c")
        sid = lax.axis_index("s")
        wid = cid * 16 + sid
        base_row = wid * TILE_ROWS
        stripe0 = sid * STRIPE

        for i in range(ZCHUNK):
            zero_v[i, :] = jnp.zeros((W8,), jnp.float32)

        for p in range(n_pass):
            for z in range(8):
                pltpu.sync_copy(zero_v,
                                acc_sh.at[pl.ds(stripe0 + z * ZCHUNK, ZCHUNK)])
            plsc.subcore_barrier()

            def body(b, _):
                r0 = base_row + b * BB
                pltpu.sync_copy(src_hbm.at[pl.ds(r0, BB)], src_v)
                pltpu.sync_copy(dst_hbm.at[pl.ds(r0, BB)], dst_v)
                if p > 0:
                    off = jnp.full((16,), p * N_PAD, jnp.int32)
                    for j in range(BB):
                        for i in range(EROW // 16):
                            idx_v[j, pl.ds(i * 16, 16)] = (
                                src_v[j, pl.ds(i * 16, 16)] + off)
                iref = src_v if p == 0 else idx_v
                cps = [
                    pltpu.async_copy(table_hbm.at[iref.at[j]],
                                     rows_v.at[pl.ds(j * EROW, EROW)], sem)
                    for j in range(BB)
                ]
                for cp in cps:
                    cp.wait()
                for j in range(BB):
                    pltpu.sync_copy(rows_v.at[pl.ds(j * EROW, EROW)],
                                    acc_sh.at[dst_v.at[j]], add=True)
                return _

            lax.fori_loop(0, NBLK, body, None)
            plsc.subcore_barrier()
            pltpu.sync_copy(acc_sh.at[pl.ds(stripe0, STRIPE)],
                            out_hbm.at[cid, p, pl.ds(stripe0, STRIPE)])

    return k


def _k1_body(dacc, xp, dis_o, xs1_o):
    deg = dacc[0, 0] + dacc[1, 0] + 1.0
    dis = lax.rsqrt(deg[:, 0:1])
    dis_o[...] = dis
    xs1_o[...] = xp[...] * dis


def _layer_body(n_pass, w_cols, acc, tbl, dis, W, b, out_o, aggregate=False,
                batch=None, sums_o=None, cnt_o=None):
    d = dis[...]
    slabs = [(acc[0, p] + acc[1, p] + tbl[p]) * d for p in range(n_pass)]
    z = jnp.concatenate(slabs, axis=1)
    h = jnp.dot(z, W[...], preferred_element_type=jnp.float32)
    h = jnp.maximum(h + b[...], 0.0)
    if aggregate:
        i = pl.program_id(0)
        oh = (lax.broadcasted_iota(jnp.int32, (NUM_GRAPHS, NB), 0)
              == batch[...]).astype(jnp.float32)
        s = jnp.dot(oh, h, preferred_element_type=jnp.float32)
        c = jnp.sum(oh, axis=1, keepdims=True)

        @pl.when(i == 0)
        def _():
            sums_o[...] = jnp.zeros_like(sums_o)
            cnt_o[...] = jnp.zeros_like(cnt_o)

        sums_o[...] += s
        cnt_o[...] += c
    else:
        hd = h * d
        for p in range(w_cols // W8):
            out_o[p] = hd[:, W8 * p:W8 * (p + 1)]


def _mlp_body(sums, cnt, fw1, fb1, fw2, fb2, out_o):
    pooled = sums[...] / jnp.maximum(cnt[...], 1.0)
    t = jnp.maximum(jnp.dot(pooled, fw1[...],
                            preferred_element_type=jnp.float32) + fb1[...], 0.0)
    out_o[...] = jnp.dot(t, fw2[...],
                         preferred_element_type=jnp.float32) + fb2[...]


def kernel(x, edge_index, batch, W1, b1, W2, b2, W3, b3,
           fc1_W, fc1_b, fc2_W, fc2_b):
    f32 = jnp.float32
    grid = N_PAD // NB

    ei = edge_index.astype(jnp.int32)
    e_extra = EROWS * EROW - N_EDGES
    src_p = jnp.concatenate(
        [ei[0], jnp.zeros((e_extra,), jnp.int32)]).reshape(EROWS, EROW)
    dst_p = jnp.concatenate(
        [ei[1], jnp.full((e_extra,), DUMP_ROW, jnp.int32)]).reshape(EROWS, EROW)
    xp = jnp.pad(x.astype(f32), ((0, N_PAD - N_NODES), (0, W8 - 4)))
    batch_p = jnp.pad(batch.astype(jnp.int32), (0, N_PAD - N_NODES),
                      constant_values=NUM_GRAPHS).reshape(1, N_PAD)
    W1p = jnp.pad(W1, ((0, W8 - 4), (0, 0)))
    ones_tbl = jnp.ones((N_PAD, W8), f32)
    b1r, b2r, b3r = b1.reshape(1, 64), b2.reshape(1, 64), b3.reshape(1, 128)
    fb1, fb2 = fc1_b.reshape(1, 64), fc2_b.reshape(1, 2)

    sc1 = _sc_agg(1)
    sc8 = _sc_agg(8)

    dacc = sc1(ones_tbl, src_p, dst_p)
    dis, xs1 = pl.pallas_call(
        _k1_body,
        grid=(grid,),
        in_specs=[
            pl.BlockSpec((2, 1, NB, W8), lambda i: (0, 0, i, 0)),
            pl.BlockSpec((NB, W8), lambda i: (i, 0)),
        ],
        out_specs=[
            pl.BlockSpec((NB, 1), lambda i: (i, 0)),
            pl.BlockSpec((NB, W8), lambda i: (i, 0)),
        ],
        out_shape=[
            jax.ShapeDtypeStruct((N_PAD, 1), f32),
            jax.ShapeDtypeStruct((N_PAD, W8), f32),
        ],
    )(dacc, xp)

    def layer_call(n_pass, w_rows, w_cols, acc, tbl, W, b):
        body = functools.partial(_layer_body, n_pass, w_cols)
        return pl.pallas_call(
            body,
            grid=(grid,),
            in_specs=[
                pl.BlockSpec((2, n_pass, NB, W8), lambda i: (0, 0, i, 0)),
                pl.BlockSpec((n_pass, NB, W8), lambda i: (0, i, 0)),
                pl.BlockSpec((NB, 1), lambda i: (i, 0)),
                pl.BlockSpec((w_rows, w_cols), lambda i: (0, 0)),
                pl.BlockSpec((1, w_cols), lambda i: (0, 0)),
            ],
            out_specs=pl.BlockSpec((w_cols // W8, NB, W8),
                                   lambda i: (0, i, 0)),
            out_shape=jax.ShapeDtypeStruct((w_cols // W8, N_PAD, W8), f32),
        )(acc, tbl, dis, W, b)

    a1 = sc1(xs1, src_p, dst_p)
    xs1r = xs1.reshape(1, N_PAD, W8)
    t2 = layer_call(1, W8, 64, a1, xs1r, W1p, b1r)

    a2 = sc8(t2.reshape(8 * N_PAD, W8), src_p, dst_p)
    t3 = layer_call(8, 64, 64, a2, t2, W2, b2r)

    a3 = sc8(t3.reshape(8 * N_PAD, W8), src_p, dst_p)

    def _k4(acc, tbl, dis_r, W, b, batch_r, sums_o, cnt_o):
        _layer_body(8, 128, acc, tbl, dis_r, W, b, None, aggregate=True,
                    batch=batch_r, sums_o=sums_o, cnt_o=cnt_o)

    sums, cnt = pl.pallas_call(
        _k4,
        grid=(grid,),
        in_specs=[
            pl.BlockSpec((2, 8, NB, W8), lambda i: (0, 0, i, 0)),
            pl.BlockSpec((8, NB, W8), lambda i: (0, i, 0)),
            pl.BlockSpec((NB, 1), lambda i: (i, 0)),
            pl.BlockSpec((64, 128), lambda i: (0, 0)),
            pl.BlockSpec((1, 128), lambda i: (0, 0)),
            pl.BlockSpec((1, NB), lambda i: (0, i)),
        ],
        out_specs=[
            pl.BlockSpec((NUM_GRAPHS, 128), lambda i: (0, 0)),
            pl.BlockSpec((NUM_GRAPHS, 1), lambda i: (0, 0)),
        ],
        out_shape=[
            jax.ShapeDtypeStruct((NUM_GRAPHS, 128), f32),
            jax.ShapeDtypeStruct((NUM_GRAPHS, 1), f32),
        ],
    )(a3, t3, dis, W3, b3r, batch_p)

    out = pl.pallas_call(
        _mlp_body,
        out_shape=jax.ShapeDtypeStruct((NUM_GRAPHS, 2), f32),
    )(sums, cnt, fc1_W, fb1, fc2_W, fb2)
    return out


# speedup vs baseline: 12.4261x; 1.6314x over previous
import functools

import jax
import jax.numpy as jnp
from jax import lax
from jax.experimental import pallas as pl
from jax.experimental.pallas import tpu as pltpu
from jax.experimental.pallas import tpu_sc as plsc

N_NODES = 100000
N_EDGES = 1600000
NUM_GRAPHS = 128

WS = 16
NB = 1024
N_PAD = 100352
STRIPE = N_PAD // 16
ZCHUNK = STRIPE // 16
NPASS_MAX = 4

NTILES = 32
EROW = 128
TILE_ROWS = 392
EROWS = NTILES * TILE_ROWS
BB = 8
NBLK = TILE_ROWS // BB
DUMP_ROW = N_PAD - 1


def _sc_agg():
    mesh = plsc.VectorSubcoreMesh(core_axis_name="c", subcore_axis_name="s")
    scratch = [
        pltpu.VMEM((BB, EROW), jnp.int32),
        pltpu.VMEM((BB, EROW), jnp.int32),
        pltpu.VMEM((BB, EROW), jnp.int32),
        pltpu.VMEM((BB * EROW, WS), jnp.float32),
        pltpu.VMEM((ZCHUNK, WS), jnp.float32),
        pltpu.VMEM((16,), jnp.int32),
        pltpu.VMEM_SHARED((N_PAD, WS), jnp.float32),
        pltpu.SemaphoreType.DMA,
    ]

    @functools.partial(
        pl.kernel,
        mesh=mesh,
        out_type=jax.ShapeDtypeStruct((2, NPASS_MAX, N_PAD, WS), jnp.float32),
        scratch_types=scratch,
        compiler_params=pltpu.CompilerParams(use_tc_tiling_on_sc=False,
                                             needs_layout_passes=False),
    )
    def k(table_hbm, src_hbm, dst_hbm, npass_hbm, zeros_hbm, out_hbm,
          src_v, dst_v, idx_v, rows_v, zero_v, npass_v, acc_sh, sem):
        cid = lax.axis_index("c")
        sid = lax.axis_index("s")
        wid = cid * 16 + sid
        base_row = wid * TILE_ROWS
        stripe0 = sid * STRIPE

        pltpu.sync_copy(zeros_hbm, zero_v)
        pltpu.sync_copy(npass_hbm, npass_v)
        npass = jnp.max(npass_v[...])

        def one_pass(p, _):
            for z in range(16):
                pltpu.sync_copy(zero_v,
                                acc_sh.at[pl.ds(stripe0 + z * ZCHUNK, ZCHUNK)])
            plsc.subcore_barrier()

            def body(b, _):
                r0 = base_row + b * BB
                pltpu.sync_copy(src_hbm.at[pl.ds(r0, BB)], src_v)
                pltpu.sync_copy(dst_hbm.at[pl.ds(r0, BB)], dst_v)
                poff = jnp.full((16,), 0, jnp.int32) + p
                for j in range(BB):
                    for i in range(EROW // 16):
                        idx_v[j, pl.ds(i * 16, 16)] = (
                            src_v[j, pl.ds(i * 16, 16)] * NPASS_MAX + poff)
                cps = [
                    pltpu.async_copy(table_hbm.at[idx_v.at[j]],
                                     rows_v.at[pl.ds(j * EROW, EROW)], sem)
                    for j in range(BB)
                ]
                for cp in cps:
                    cp.wait()
                for j in range(BB):
                    pltpu.sync_copy(rows_v.at[pl.ds(j * EROW, EROW)],
                                    acc_sh.at[dst_v.at[j]], add=True)
                return _

            lax.fori_loop(0, NBLK, body, None)
            plsc.subcore_barrier()
            pltpu.sync_copy(acc_sh.at[pl.ds(stripe0, STRIPE)],
                            out_hbm.at[cid, p, pl.ds(stripe0, STRIPE)])
            return _

        lax.fori_loop(0, npass, one_pass, None)

    return k


def _k1_body(dacc, xp, dis_o, xs1_o):
    deg = dacc[0, 0] + dacc[1, 0] + 1.0
    dis = lax.rsqrt(deg[:, 0:1])
    dis_o[...] = dis
    xs1_o[...] = xp[...] * dis


def _layer_body(n_pass, acc, tbl, dis, W, b, out_o, aggregate=False,
                batch=None, sums_o=None, cnt_o=None):
    d = dis[...]
    slabs = [(acc[0, p] + acc[1, p]) * d for p in range(n_pass)]
    z = jnp.concatenate(slabs, axis=1) if n_pass > 1 else slabs[0]
    z = z + tbl[:, :n_pass * WS] * d
    h = jnp.dot(z, W[...], preferred_element_type=jnp.float32)
    h = jnp.maximum(h + b[...], 0.0)
    if aggregate:
        i = pl.program_id(0)
        oh = (lax.broadcasted_iota(jnp.int32, (NUM_GRAPHS, NB), 0)
              == batch[...]).astype(jnp.float32)
        s = jnp.dot(oh, h, preferred_element_type=jnp.float32)
        c = jnp.sum(oh, axis=1, keepdims=True)

        @pl.when(i == 0)
        def _():
            sums_o[...] = jnp.zeros_like(sums_o)
            cnt_o[...] = jnp.zeros_like(cnt_o)

        sums_o[...] += s
        cnt_o[...] += c
    else:
        out_o[...] = h * d


def _mlp_body(sums, cnt, fw1, fb1, fw2, fb2, out_o):
    pooled = sums[...] / jnp.maximum(cnt[...], 1.0)
    t = jnp.maximum(jnp.dot(pooled, fw1[...],
                            preferred_element_type=jnp.float32) + fb1[...], 0.0)
    out_o[...] = jnp.dot(t, fw2[...],
                         preferred_element_type=jnp.float32) + fb2[...]


def kernel(x, edge_index, batch, W1, b1, W2, b2, W3, b3,
           fc1_W, fc1_b, fc2_W, fc2_b):
    f32 = jnp.float32
    grid = N_PAD // NB

    ei = edge_index.astype(jnp.int32)
    e_extra = EROWS * EROW - N_EDGES
    src_p = jnp.concatenate(
        [ei[0], jnp.zeros((e_extra,), jnp.int32)]).reshape(EROWS, EROW)
    dst_p = jnp.concatenate(
        [ei[1], jnp.full((e_extra,), DUMP_ROW, jnp.int32)]).reshape(EROWS, EROW)
    xp = jnp.pad(x.astype(f32), ((0, N_PAD - N_NODES), (0, 60)))
    batch_p = jnp.pad(batch.astype(jnp.int32), (0, N_PAD - N_NODES),
                      constant_values=NUM_GRAPHS).reshape(1, N_PAD)
    W1p = jnp.pad(W1, ((0, WS - 4), (0, 0)))
    ones_tbl = jnp.ones((NPASS_MAX * N_PAD, WS), f32)
    zeros_sm = jnp.zeros((ZCHUNK, WS), f32)
    one_pass = jnp.ones((16,), jnp.int32)
    four_pass = jnp.full((16,), NPASS_MAX, jnp.int32)
    b1r, b2r, b3r = b1.reshape(1, 64), b2.reshape(1, 64), b3.reshape(1, 128)
    fb1, fb2 = fc1_b.reshape(1, 64), fc2_b.reshape(1, 2)

    sc = _sc_agg()

    dacc = sc(ones_tbl, src_p, dst_p, one_pass, zeros_sm)
    dis, xs1 = pl.pallas_call(
        _k1_body,
        grid=(grid,),
        in_specs=[
            pl.BlockSpec((2, 1, NB, WS), lambda i: (0, 0, i, 0)),
            pl.BlockSpec((NB, 64), lambda i: (i, 0)),
        ],
        out_specs=[
            pl.BlockSpec((NB, 1), lambda i: (i, 0)),
            pl.BlockSpec((NB, 64), lambda i: (i, 0)),
        ],
        out_shape=[
            jax.ShapeDtypeStruct((N_PAD, 1), f32),
            jax.ShapeDtypeStruct((N_PAD, 64), f32),
        ],
    )(dacc, xp)

    def layer_call(n_pass, w_rows, w_cols, acc, tbl, W, b):
        body = functools.partial(_layer_body, n_pass)
        return pl.pallas_call(
            body,
            grid=(grid,),
            in_specs=[
                pl.BlockSpec((2, n_pass, NB, WS), lambda i: (0, 0, i, 0)),
                pl.BlockSpec((NB, 64), lambda i: (i, 0)),
                pl.BlockSpec((NB, 1), lambda i: (i, 0)),
                pl.BlockSpec((w_rows, w_cols), lambda i: (0, 0)),
                pl.BlockSpec((1, w_cols), lambda i: (0, 0)),
            ],
            out_specs=pl.BlockSpec((NB, w_cols), lambda i: (i, 0)),
            out_shape=jax.ShapeDtypeStruct((N_PAD, w_cols), f32),
        )(acc, tbl, dis, W, b)

    a1 = sc(xs1.reshape(NPASS_MAX * N_PAD, WS), src_p, dst_p, one_pass, zeros_sm)
    t2 = layer_call(1, WS, 64, a1, xs1, W1p, b1r)

    a2 = sc(t2.reshape(NPASS_MAX * N_PAD, WS), src_p, dst_p, four_pass, zeros_sm)
    t3 = layer_call(NPASS_MAX, 64, 64, a2, t2, W2, b2r)

    a3 = sc(t3.reshape(NPASS_MAX * N_PAD, WS), src_p, dst_p, four_pass, zeros_sm)

    def _k4(acc, tbl, dis_r, W, b, batch_r, sums_o, cnt_o):
        _layer_body(NPASS_MAX, acc, tbl, dis_r, W, b, None, aggregate=True,
                    batch=batch_r, sums_o=sums_o, cnt_o=cnt_o)

    sums, cnt = pl.pallas_call(
        _k4,
        grid=(grid,),
        in_specs=[
            pl.BlockSpec((2, NPASS_MAX, NB, WS), lambda i: (0, 0, i, 0)),
            pl.BlockSpec((NB, 64), lambda i: (i, 0)),
            pl.BlockSpec((NB, 1), lambda i: (i, 0)),
            pl.BlockSpec((64, 128), lambda i: (0, 0)),
            pl.BlockSpec((1, 128), lambda i: (0, 0)),
            pl.BlockSpec((1, NB), lambda i: (0, i)),
        ],
        out_specs=[
            pl.BlockSpec((NUM_GRAPHS, 128), lambda i: (0, 0)),
            pl.BlockSpec((NUM_GRAPHS, 1), lambda i: (0, 0)),
        ],
        out_shape=[
            jax.ShapeDtypeStruct((NUM_GRAPHS, 128), f32),
            jax.ShapeDtypeStruct((NUM_GRAPHS, 1), f32),
        ],
    )(a3, t3, dis, W3, b3r, batch_p)

    out = pl.pallas_call(
        _mlp_body,
        out_shape=jax.ShapeDtypeStruct((NUM_GRAPHS, 2), f32),
    )(sums, cnt, fc1_W, fb1, fc2_W, fb2)
    return out
